# Initial kernel scaffold; baseline (speedup 1.0000x reference)
import jax
import jax.numpy as jnp
from jax.experimental import pallas as pl


def kernel(x1, f1, edge_index1, g1, x2, f2, edge_index2, g2, Wx, bx, Wf1, bf1, Wf2, bf2, W1, b1, W2, b2, W3, b3, W4, b4):
    raise NotImplementedError("write your pallas kernel here")


# speedup vs baseline: 10.9838x; 10.9838x over previous
import functools

import jax
import jax.numpy as jnp
from jax import lax
from jax.experimental import pallas as pl
from jax.experimental.pallas import tpu as pltpu
from jax.experimental.pallas import tpu_sc as plsc

NTILES = 32
NSUB = 16
CH = 128


def _ceil_div(a, b):
    return -(-a // b)


@functools.lru_cache(maxsize=None)
def _sc_scatter(w, k_chunks, np_rows):
    rpt = np_rows // NSUB
    zb = rpt // CH
    mesh = plsc.VectorSubcoreMesh(core_axis_name="c", subcore_axis_name="s")

    @functools.partial(
        pl.kernel,
        out_type=jax.ShapeDtypeStruct((2, np_rows, w), jnp.float32),
        mesh=mesh,
        scratch_types=[
            pltpu.VMEM((CH,), jnp.int32),
            pltpu.VMEM((CH,), jnp.int32),
            pltpu.VMEM((CH, w), jnp.float32),
            pltpu.VMEM((CH, w), jnp.float32),
            pltpu.VMEM_SHARED((np_rows, w), jnp.float32),
            pltpu.SemaphoreType.DMA,
        ],
        compiler_params=pltpu.CompilerParams(use_tc_tiling_on_sc=False),
    )
    def kern(src_hbm, dst_hbm, u_hbm, zeros_hbm, out_hbm,
             sidx, didx, rows, zbuf, acc, sem):
        cid = lax.axis_index("c")
        sid = lax.axis_index("s")
        wid = cid * NSUB + sid
        pltpu.sync_copy(zeros_hbm, zbuf)

        def zsec(i, carry):
            pltpu.sync_copy(zbuf, acc.at[pl.ds(sid * rpt + i * CH, CH)])
            return carry

        lax.fori_loop(0, zb, zsec, 0)
        plsc.subcore_barrier()

        def step(j, carry):
            pltpu.sync_copy(src_hbm.at[wid, j], sidx)
            pltpu.sync_copy(dst_hbm.at[wid, j], didx)
            pltpu.async_copy(u_hbm.at[sidx], rows, sem).wait()
            pltpu.sync_copy(rows, acc.at[didx], add=True)
            return carry

        lax.fori_loop(0, k_chunks, step, 0)
        plsc.subcore_barrier()
        pltpu.sync_copy(acc.at[pl.ds(sid * rpt, rpt)],
                        out_hbm.at[cid, pl.ds(sid * rpt, rpt)])

    return kern


@functools.lru_cache(maxsize=None)
def _sc_degree(k_chunks, np2_rows):
    rpt = np2_rows // NSUB
    zb = rpt // CH
    mesh = plsc.VectorSubcoreMesh(core_axis_name="c", subcore_axis_name="s")

    @functools.partial(
        pl.kernel,
        out_type=jax.ShapeDtypeStruct((2, np2_rows, 8), jnp.float32),
        mesh=mesh,
        scratch_types=[
            pltpu.VMEM((CH,), jnp.int32),
            pltpu.VMEM((CH, 8), jnp.float32),
            pltpu.VMEM((CH, 8), jnp.float32),
            pltpu.VMEM_SHARED((np2_rows, 8), jnp.float32),
        ],
        compiler_params=pltpu.CompilerParams(use_tc_tiling_on_sc=False),
    )
    def kern(dst_hbm, zeros_hbm, ones_hbm, out_hbm, didx, ones, zbuf, acc):
        cid = lax.axis_index("c")
        sid = lax.axis_index("s")
        wid = cid * NSUB + sid
        pltpu.sync_copy(zeros_hbm, zbuf)
        pltpu.sync_copy(ones_hbm, ones)

        def zsec(i, carry):
            pltpu.sync_copy(zbuf, acc.at[pl.ds(sid * rpt + i * CH, CH)])
            return carry

        lax.fori_loop(0, zb, zsec, 0)
        plsc.subcore_barrier()

        def step(j, carry):
            pltpu.sync_copy(dst_hbm.at[wid, j], didx)
            pltpu.sync_copy(ones, acc.at[didx], add=True)
            return carry

        lax.fori_loop(0, k_chunks, step, 0)
        plsc.subcore_barrier()
        pltpu.sync_copy(acc.at[pl.ds(sid * rpt, rpt)],
                        out_hbm.at[cid, pl.ds(sid * rpt, rpt)])

    return kern


def _pack_idx(a, k_chunks, fill):
    total = NTILES * k_chunks * CH
    pad = total - a.shape[0]
    a = jnp.concatenate([a, jnp.full((pad,), fill, a.dtype)])
    return a.reshape(NTILES, k_chunks, CH)


def _row_spec(r, w):
    return pl.BlockSpec((r, w), lambda i: (i, 0))


def _full_spec(shape):
    return pl.BlockSpec(shape, lambda i: tuple(0 for _ in shape))


def _par_spec(r, w):
    return pl.BlockSpec((2, r, w), lambda i: (0, i, 0))


def _relu(v):
    return jnp.maximum(v, 0.0)


def _dot(a, b):
    return jnp.dot(a, b, preferred_element_type=jnp.float32)


def _tc_prep(x, f, degp, Wx, bx, Wf1, bf1, Wf2, bf2, g, n, r, grid, np_rows):
    blocks_per_graph = np_rows // r

    def body(x_ref, f_ref, dg_ref, wx_ref, bx_ref, wf1_ref, bf1_ref,
             wf2_ref, bf2_ref, h_ref, u0a_ref, u0b_ref, dinv_ref):
        deg = 1.0 + dg_ref[0, :, 0:1] + dg_ref[1, :, 0:1]
        dinv = lax.rsqrt(deg)
        x0 = _relu(_dot(x_ref[...], wx_ref[...]) + bx_ref[...])
        f1 = _relu(_dot(f_ref[...], wf1_ref[...]) + bf1_ref[...])
        f0 = _relu(_dot(f1, wf2_ref[...]) + bf2_ref[...])
        h_ref[:, 0:32] = x0
        h_ref[:, 32:48] = f0
        u0a_ref[...] = dinv * x0
        u0b_ref[...] = dinv * f0
        dinv_ref[...] = dinv

    return pl.pallas_call(
        body,
        grid=(grid,),
        in_specs=[
            _row_spec(r, 3),
            _row_spec(r, 128),
            pl.BlockSpec((2, r, 8), lambda i: (0, g * blocks_per_graph + i, 0)),
            _full_spec((3, 32)), _full_spec((1, 32)),
            _full_spec((128, 64)), _full_spec((1, 64)),
            _full_spec((64, 16)), _full_spec((1, 16)),
        ],
        out_specs=[
            _row_spec(r, 48), _row_spec(r, 32), _row_spec(r, 16),
            _row_spec(r, 1),
        ],
        out_shape=[
            jax.ShapeDtypeStruct((n, 48), jnp.float32),
            jax.ShapeDtypeStruct((n, 32), jnp.float32),
            jax.ShapeDtypeStruct((n, 16), jnp.float32),
            jax.ShapeDtypeStruct((n, 1), jnp.float32),
        ],
    )(x, f, degp, Wx, bx.reshape(1, -1), Wf1, bf1.reshape(1, -1),
      Wf2, bf2.reshape(1, -1))


def _tc_layer1(dinv, h0, s0a, s0b, W1, b1, n, r, grid):
    def body(dv_ref, h0_ref, sa_ref, sb_ref, w_ref, b_ref,
             a0_ref, h1_ref, u1_ref):
        d = dv_ref[...]
        d2 = d * d
        a0a = d * (sa_ref[0] + sa_ref[1]) + d2 * h0_ref[:, 0:32]
        a0b = d * (sb_ref[0] + sb_ref[1]) + d2 * h0_ref[:, 32:48]
        a0_ref[:, 0:32] = a0a
        a0_ref[:, 32:48] = a0b
        h1 = _relu(_dot(a0a, w_ref[0:32, :]) + _dot(a0b, w_ref[32:48, :])
                   + b_ref[...])
        h1_ref[...] = h1
        u1_ref[...] = d * h1

    return pl.pallas_call(
        body,
        grid=(grid,),
        in_specs=[
            _row_spec(r, 1), _row_spec(r, 48),
            _par_spec(r, 32), _par_spec(r, 16),
            _full_spec((48, 32)), _full_spec((1, 32)),
        ],
        out_specs=[_row_spec(r, 48), _row_spec(r, 32), _row_spec(r, 32)],
        out_shape=[
            jax.ShapeDtypeStruct((n, 48), jnp.float32),
            jax.ShapeDtypeStruct((n, 32), jnp.float32),
            jax.ShapeDtypeStruct((n, 32), jnp.float32),
        ],
    )(dinv, h0, s0a, s0b, W1, b1.reshape(1, -1))


def _tc_layer_mid(dinv, h_prev, s_prev, a_list, W, b, n, r, grid):
    na = len(a_list)
    widths = [a.shape[1] for a in a_list]

    def body(*refs):
        dv_ref, hp_ref, s_ref = refs[0], refs[1], refs[2]
        a_refs = refs[3:3 + na]
        w_ref, b_ref = refs[3 + na], refs[4 + na]
        ak_ref, hn_ref, un_ref = refs[5 + na], refs[6 + na], refs[7 + na]
        d = dv_ref[...]
        d2 = d * d
        ak = d * (s_ref[0] + s_ref[1]) + d2 * hp_ref[...]
        ak_ref[...] = ak
        off = 0
        acc = b_ref[...]
        for a_ref, wd in zip(a_refs, widths):
            acc = acc + _dot(a_ref[...], w_ref[off:off + wd, :])
            off += wd
        acc = acc + _dot(ak, w_ref[off:off + ak.shape[1], :])
        hn = _relu(acc)
        hn_ref[...] = hn
        un_ref[...] = d * hn

    wk = W.shape[0]
    return pl.pallas_call(
        body,
        grid=(grid,),
        in_specs=[
            _row_spec(r, 1), _row_spec(r, 32), _par_spec(r, 32),
            *[_row_spec(r, wd) for wd in widths],
            _full_spec((wk, 32)), _full_spec((1, 32)),
        ],
        out_specs=[_row_spec(r, 32), _row_spec(r, 32), _row_spec(r, 32)],
        out_shape=[
            jax.ShapeDtypeStruct((n, 32), jnp.float32),
            jax.ShapeDtypeStruct((n, 32), jnp.float32),
            jax.ShapeDtypeStruct((n, 32), jnp.float32),
        ],
    )(dinv, h_prev, s_prev, *a_list, W, b.reshape(1, -1))


def _tc_layer4(dinv, h3, s3, a0, a1, a2, W4, b4, n, r, grid):
    def body(dv_ref, h3_ref, s_ref, a0_ref, a1_ref, a2_ref, w_ref, b_ref,
             out_ref):
        d = dv_ref[...]
        d2 = d * d
        a3 = d * (s_ref[0] + s_ref[1]) + d2 * h3_ref[...]
        acc = (b_ref[...]
               + _dot(a0_ref[...], w_ref[0:48, :])
               + _dot(a1_ref[...], w_ref[48:80, :])
               + _dot(a2_ref[...], w_ref[80:112, :])
               + _dot(a3, w_ref[112:144, :]))
        out_ref[...] = _relu(acc)

    return pl.pallas_call(
        body,
        grid=(grid,),
        in_specs=[
            _row_spec(r, 1), _row_spec(r, 32), _par_spec(r, 32),
            _row_spec(r, 48), _row_spec(r, 32), _row_spec(r, 32),
            _full_spec((144, 64)), _full_spec((1, 64)),
        ],
        out_specs=_row_spec(r, 64),
        out_shape=jax.ShapeDtypeStruct((n, 64), jnp.float32),
    )(dinv, h3, s3, a0, a1, a2, W4, b4.reshape(1, -1))


def kernel(x1, f1, edge_index1, g1, x2, f2, edge_index2, g2,
           Wx, bx, Wf1, bf1, Wf2, bf2, W1, b1, W2, b2, W3, b3, W4, b4):
    n = x1.shape[0]
    e = edge_index1.shape[1]
    np_rows = _ceil_div(n, NSUB * CH) * NSUB * CH
    r = np_rows // NSUB
    grid = NSUB
    k_lay = _ceil_div(e, NTILES * CH)
    k_deg = _ceil_div(2 * e, NTILES * CH)

    src1 = _pack_idx(edge_index1[0], k_lay, 0)
    dst1 = _pack_idx(edge_index1[1], k_lay, np_rows - 1)
    src2 = _pack_idx(edge_index2[0], k_lay, 0)
    dst2 = _pack_idx(edge_index2[1], k_lay, np_rows - 1)
    dst_both = _pack_idx(
        jnp.concatenate([edge_index1[1], edge_index2[1] + np_rows]),
        k_deg, 2 * np_rows - 1)

    zeros8 = jnp.zeros((CH, 8), jnp.float32)
    ones8 = jnp.ones((CH, 8), jnp.float32)
    zeros16 = jnp.zeros((CH, 16), jnp.float32)
    zeros32 = jnp.zeros((CH, 32), jnp.float32)

    degp = _sc_degree(k_deg, 2 * np_rows)(dst_both, zeros8, ones8)

    _scat32 = _sc_scatter(32, k_lay, np_rows)
    _scat16 = _sc_scatter(16, k_lay, np_rows)
    scat32 = lambda s, d, u: _scat32(s, d, u, zeros32)
    scat16 = lambda s, d, u: _scat16(s, d, u, zeros16)

    outs = []
    for gg, (x, f, srcp, dstp) in enumerate(
            [(x1, f1, src1, dst1), (x2, f2, src2, dst2)]):
        h0, u0a, u0b, dinv = _tc_prep(
            x, f, degp, Wx, bx, Wf1, bf1, Wf2, bf2,
            gg, n, r, grid, np_rows)
        s0a = scat32(srcp, dstp, u0a)
        s0b = scat16(srcp, dstp, u0b)
        a0, h1, u1 = _tc_layer1(dinv, h0, s0a, s0b, W1, b1, n, r, grid)
        s1 = scat32(srcp, dstp, u1)
        a1, h2, u2 = _tc_layer_mid(dinv, h1, s1, [a0], W2, b2, n, r, grid)
        s2 = scat32(srcp, dstp, u2)
        a2, h3, u3 = _tc_layer_mid(dinv, h2, s2, [a0, a1], W3, b3, n, r, grid)
        s3 = scat32(srcp, dstp, u3)
        h4 = _tc_layer4(dinv, h3, s3, a0, a1, a2, W4, b4, n, r, grid)
        outs.append(h4)

    return (outs[0], outs[1], g1, g2)


# speedup vs baseline: 26.5393x; 2.4162x over previous
import functools

import jax
import jax.numpy as jnp
from jax import lax
from jax.experimental import pallas as pl
from jax.experimental.pallas import tpu as pltpu
from jax.experimental.pallas import tpu_sc as plsc

NSUB = 16
CH = 128
NB = 4
SB = 8
NBD = 8


def _ceil_div(a, b):
    return -(-a // b)


@functools.lru_cache(maxsize=None)
def _sc_scatter(w, k_chunks, np_rows):
    rpt = np_rows // NSUB
    sblocks = k_chunks // SB
    mesh = plsc.VectorSubcoreMesh(core_axis_name="c", subcore_axis_name="s")

    @functools.partial(
        pl.kernel,
        out_type=jax.ShapeDtypeStruct((2, np_rows, w), jnp.float32),
        mesh=mesh,
        scratch_types=[
            pltpu.VMEM((SB, 2, CH), jnp.int32),
            pltpu.VMEM((NB, CH, w), jnp.float32),
            pltpu.VMEM_SHARED((np_rows, w), jnp.float32),
            pltpu.SemaphoreType.DMA((SB,)),
            pltpu.SemaphoreType.DMA((NB,)),
            pltpu.SemaphoreType.DMA((NB,)),
        ],
        compiler_params=pltpu.CompilerParams(use_tc_tiling_on_sc=False),
    )
    def kern(idx_hbm, u_hbm, zeros_hbm, out_hbm, idxr, rows, acc,
             isem, gsem, ssem):
        cid = lax.axis_index("c")
        sid = lax.axis_index("s")
        pltpu.sync_copy(zeros_hbm, acc.at[pl.ds(sid * rpt, rpt)])
        plsc.subcore_barrier()

        def idx_load(c, slot):
            pltpu.async_copy(idx_hbm.at[cid, sid, c], idxr.at[slot],
                             isem.at[slot])

        def idx_wait(c, slot):
            pltpu.make_async_copy(idx_hbm.at[cid, sid, c], idxr.at[slot],
                                  isem.at[slot]).wait()

        def gather_start(islot, rslot):
            pltpu.async_copy(u_hbm.at[idxr.at[islot, 0]], rows.at[rslot],
                             gsem.at[rslot])

        def gather_wait(islot, rslot):
            pltpu.make_async_copy(u_hbm.at[idxr.at[islot, 0]],
                                  rows.at[rslot], gsem.at[rslot]).wait()

        def scatter_start(islot, rslot):
            pltpu.async_copy(rows.at[rslot], acc.at[idxr.at[islot, 1]],
                             ssem.at[rslot], add=True)

        def scatter_wait(islot, rslot):
            pltpu.make_async_copy(rows.at[rslot], acc.at[idxr.at[islot, 1]],
                                  ssem.at[rslot]).wait()

        for j in range(SB):
            idx_load(j, j)
        for j in range(NB):
            idx_wait(j, j)
            gather_start(j, j)

        def sblock(sg, carry):
            base = sg * SB
            for j in range(NB):
                gather_wait(j, j)
                scatter_start(j, j)
            for j in range(NB):
                scatter_wait(j, j)
                idx_wait(base + NB + j, NB + j)
                gather_start(NB + j, j)

                @pl.when(base + SB + j < k_chunks)
                def _():
                    idx_load(base + SB + j, j)
            for j in range(NB):
                gather_wait(NB + j, j)
                scatter_start(NB + j, j)
            for j in range(NB):
                scatter_wait(NB + j, j)

                @pl.when(base + SB + j < k_chunks)
                def _():
                    idx_wait(base + SB + j, j)
                    gather_start(j, j)
                    pl.when(base + SB + NB + j < k_chunks)(
                        lambda: idx_load(base + SB + NB + j, NB + j))
            return carry

        lax.fori_loop(0, sblocks, sblock, 0)
        plsc.subcore_barrier()
        pltpu.sync_copy(acc.at[pl.ds(sid * rpt, rpt)],
                        out_hbm.at[cid, pl.ds(sid * rpt, rpt)])

    return kern


@functools.lru_cache(maxsize=None)
def _sc_degree(k_chunks, np_rows):
    rpt = np_rows // NSUB
    groups = k_chunks // NBD
    mesh = plsc.VectorSubcoreMesh(core_axis_name="c", subcore_axis_name="s")

    @functools.partial(
        pl.kernel,
        out_type=jax.ShapeDtypeStruct((2, np_rows, 8), jnp.float32),
        mesh=mesh,
        scratch_types=[
            pltpu.VMEM((k_chunks, CH), jnp.int32),
            pltpu.VMEM((CH, 8), jnp.float32),
            pltpu.VMEM_SHARED((np_rows, 8), jnp.float32),
            pltpu.SemaphoreType.DMA((NBD,)),
        ],
        compiler_params=pltpu.CompilerParams(use_tc_tiling_on_sc=False),
    )
    def kern(idx_hbm, zeros_hbm, ones_hbm, out_hbm, idxall, ones, acc, ssem):
        cid = lax.axis_index("c")
        sid = lax.axis_index("s")
        pltpu.sync_copy(idx_hbm.at[cid, sid], idxall)
        pltpu.sync_copy(ones_hbm, ones)
        pltpu.sync_copy(zeros_hbm, acc.at[pl.ds(sid * rpt, rpt)])
        plsc.subcore_barrier()

        def group(g, carry):
            base = g * NBD
            for b in range(NBD):
                @pl.when(g > 0)
                def _():
                    pltpu.make_async_copy(ones, acc.at[idxall.at[base + b]],
                                          ssem.at[b]).wait()
                pltpu.async_copy(ones, acc.at[idxall.at[base + b]],
                                 ssem.at[b], add=True)
            return carry

        lax.fori_loop(0, groups, group, 0)
        for b in range(NBD):
            pltpu.make_async_copy(ones, acc.at[idxall.at[b]],
                                  ssem.at[b]).wait()
        plsc.subcore_barrier()
        pltpu.sync_copy(acc.at[pl.ds(sid * rpt, rpt)],
                        out_hbm.at[cid, pl.ds(sid * rpt, rpt)])

    return kern


def _pack_pairs(src, dst, k_chunks, np_rows):
    total = NSUB * k_chunks * CH
    out = []
    for g, (s, d) in enumerate(zip(src, dst)):
        pad = total - s.shape[0]
        s = jnp.concatenate([s + g * np_rows,
                             jnp.full((pad,), g * np_rows, s.dtype)])
        d = jnp.concatenate([d, jnp.full((pad,), np_rows - 1, d.dtype)])
        out.append(jnp.stack([s.reshape(NSUB, k_chunks, CH),
                              d.reshape(NSUB, k_chunks, CH)], axis=2))
    return jnp.stack(out)


def _pack_dst(dst, k_chunks, np_rows):
    total = NSUB * k_chunks * CH
    out = []
    for d in dst:
        pad = total - d.shape[0]
        d = jnp.concatenate([d, jnp.full((pad,), np_rows - 1, d.dtype)])
        out.append(d.reshape(NSUB, k_chunks, CH))
    return jnp.stack(out)


def _full(shape):
    return pl.BlockSpec(shape, lambda g, i: tuple(0 for _ in shape))


def _relu(v):
    return jnp.maximum(v, 0.0)


def _dot(a, b):
    return jnp.dot(a, b, preferred_element_type=jnp.float32)


def _tc_prep(x, f, degp, Wx, bx, Wf1, bf1, Wf2, bf2, n, r, np_rows):
    def body(x_ref, f_ref, dg_ref, wx_ref, bx_ref, wf1_ref, bf1_ref,
             wf2_ref, bf2_ref, h_ref, u0a_ref, u0b_ref, dinv_ref):
        deg = 1.0 + dg_ref[0, :, 0:1]
        dinv = lax.rsqrt(deg)
        x0 = _relu(_dot(x_ref[0], wx_ref[...]) + bx_ref[...])
        f1 = _relu(_dot(f_ref[0], wf1_ref[...]) + bf1_ref[...])
        f0 = _relu(_dot(f1, wf2_ref[...]) + bf2_ref[...])
        h_ref[0, :, 0:32] = x0
        h_ref[0, :, 32:48] = f0
        u0a_ref[...] = dinv * x0
        u0b_ref[...] = dinv * f0
        dinv_ref[0] = dinv

    def spec3(w):
        return pl.BlockSpec((1, r, w), lambda g, i: (g, i, 0))

    return pl.pallas_call(
        body,
        grid=(2, NSUB),
        in_specs=[
            spec3(3), spec3(128), spec3(8),
            _full((3, 32)), _full((1, 32)),
            _full((128, 64)), _full((1, 64)),
            _full((64, 16)), _full((1, 16)),
        ],
        out_specs=[
            spec3(48),
            pl.BlockSpec((r, 32), lambda g, i: (g * NSUB + i, 0)),
            pl.BlockSpec((r, 16), lambda g, i: (g * NSUB + i, 0)),
            spec3(1),
        ],
        out_shape=[
            jax.ShapeDtypeStruct((2, np_rows, 48), jnp.float32),
            jax.ShapeDtypeStruct((2 * np_rows, 32), jnp.float32),
            jax.ShapeDtypeStruct((2 * np_rows, 16), jnp.float32),
            jax.ShapeDtypeStruct((2, np_rows, 1), jnp.float32),
        ],
    )(x, f, degp, Wx, bx.reshape(1, -1), Wf1, bf1.reshape(1, -1),
      Wf2, bf2.reshape(1, -1))


def _tc_layer(dinv, h_prev, s_list, a_list, W, b, wout, r, np_rows,
              last=False):
    na = len(a_list)
    ns = len(s_list)
    s_widths = [s.shape[2] for s in s_list]
    a_widths = [a.shape[2] for a in a_list]
    wk = W.shape[0]

    def body(*refs):
        dv_ref, hp_ref = refs[0], refs[1]
        s_refs = refs[2:2 + ns]
        a_refs = refs[2 + ns:2 + ns + na]
        w_ref, b_ref = refs[2 + ns + na], refs[3 + ns + na]
        orefs = refs[4 + ns + na:]
        d = dv_ref[0]
        d2 = d * d
        off = 0
        aks = []
        for s_ref, wd in zip(s_refs, s_widths):
            aks.append(d * s_ref[0] + d2 * hp_ref[0, :, off:off + wd])
            off += wd
        acc = b_ref[...]
        off = 0
        for a_ref, wd in zip(a_refs, a_widths):
            acc = acc + _dot(a_ref[0], w_ref[off:off + wd, :])
            off += wd
        for ak in aks:
            acc = acc + _dot(ak, w_ref[off:off + ak.shape[1], :])
            off += ak.shape[1]
        hn = _relu(acc)
        if last:
            orefs[0][0] = hn
        else:
            ak_ref, hn_ref, un_ref = orefs
            aoff = 0
            for ak, wd in zip(aks, s_widths):
                ak_ref[0, :, aoff:aoff + wd] = ak
                aoff += wd
            hn_ref[0] = hn
            un_ref[...] = d * hn

    hw = h_prev.shape[2]

    def spec3(w):
        return pl.BlockSpec((1, r, w), lambda g, i: (g, i, 0))

    in_specs = ([spec3(1), spec3(hw)]
                + [spec3(wd) for wd in s_widths]
                + [spec3(wd) for wd in a_widths]
                + [_full((wk, wout)), _full((1, wout))])
    if last:
        out_specs = spec3(wout)
        out_shape = jax.ShapeDtypeStruct((2, np_rows, wout), jnp.float32)
    else:
        out_specs = [
            spec3(hw),
            spec3(wout),
            pl.BlockSpec((r, wout), lambda g, i: (g * NSUB + i, 0)),
        ]
        out_shape = [
            jax.ShapeDtypeStruct((2, np_rows, hw), jnp.float32),
            jax.ShapeDtypeStruct((2, np_rows, wout), jnp.float32),
            jax.ShapeDtypeStruct((2 * np_rows, wout), jnp.float32),
        ]
    return pl.pallas_call(
        body,
        grid=(2, NSUB),
        in_specs=in_specs,
        out_specs=out_specs,
        out_shape=out_shape,
    )(dinv, h_prev, *s_list, *a_list, W, b.reshape(1, -1))


def kernel(x1, f1, edge_index1, g1, x2, f2, edge_index2, g2,
           Wx, bx, Wf1, bf1, Wf2, bf2, W1, b1, W2, b2, W3, b3, W4, b4):
    n = x1.shape[0]
    e = edge_index1.shape[1]
    np_rows = _ceil_div(n, NSUB * CH) * NSUB * CH
    r = np_rows // NSUB
    k_ch = _ceil_div(e, NSUB * CH * SB) * SB

    idx_pairs = _pack_pairs((edge_index1[0], edge_index2[0]),
                            (edge_index1[1], edge_index2[1]), k_ch, np_rows)
    k_deg = _ceil_div(e, NSUB * CH * NBD) * NBD
    idx_deg = _pack_dst((edge_index1[1], edge_index2[1]), k_deg, np_rows)

    zeros8 = jnp.zeros((r, 8), jnp.float32)
    ones8 = jnp.ones((CH, 8), jnp.float32)
    zeros16 = jnp.zeros((r, 16), jnp.float32)
    zeros32 = jnp.zeros((r, 32), jnp.float32)

    x = jnp.stack([x1, x2])
    f = jnp.stack([f1, f2])

    degp = _sc_degree(k_deg, np_rows)(idx_deg, zeros8, ones8)

    scat32 = _sc_scatter(32, k_ch, np_rows)
    scat16 = _sc_scatter(16, k_ch, np_rows)

    h0, u0a, u0b, dinv = _tc_prep(x, f, degp, Wx, bx, Wf1, bf1, Wf2, bf2,
                                  n, r, np_rows)
    s0a = scat32(idx_pairs, u0a, zeros32)
    s0b = scat16(idx_pairs, u0b, zeros16)
    a0, h1, u1 = _tc_layer(dinv, h0, [s0a, s0b], [], W1, b1, 32, r, np_rows)
    s1 = scat32(idx_pairs, u1, zeros32)
    a1, h2, u2 = _tc_layer(dinv, h1, [s1], [a0], W2, b2, 32, r, np_rows)
    s2 = scat32(idx_pairs, u2, zeros32)
    a2, h3, u3 = _tc_layer(dinv, h2, [s2], [a0, a1], W3, b3, 32, r, np_rows)
    s3 = scat32(idx_pairs, u3, zeros32)
    h4 = _tc_layer(dinv, h3, [s3], [a0, a1, a2], W4, b4, 64, r, np_rows,
                   last=True)

    return (h4[0, :n], h4[1, :n], g1, g2)
